# baseline (speedup 1.0000x reference)
import functools

import jax
import jax.numpy as jnp
from jax import lax
from jax.experimental import pallas as pl
from jax.experimental.pallas import tpu as pltpu
from jax.experimental.pallas import tpu_sc as plsc

KTOP = 100
ROWS = 4096
COLS = 16384
LANES = 16
NV = COLS // LANES
NC = 2
NS = 16
NW = NC * NS
ROWS_PER_W = ROWS // NW
SAMPLE_STRIDE = 8
QLO = 13
CSTRIDE = 1025
JCAP = 32


def _sortable(xv):
    iv = lax.bitcast_convert_type(xv, jnp.int32)
    return jnp.where(iv < 0, iv ^ 0x7FFFFFFF, iv)


def _gat(vec, idx_splat):
    return vec.at[idx_splat].get(mode="promise_in_bounds")


def _popc(mask):
    return plsc.all_reduce_population_count(mask)


def _scan_fixed(hist_v, start_vreg, k_splat, iota, niter):
    zerov = jnp.zeros((LANES,), jnp.int32)

    def body(j, st):
        cum, b, cg, cb, found = st
        i = jnp.maximum(start_vreg - j, 0)
        v = hist_v[pl.ds(i * LANES, LANES)]
        sfx = lax.rev(plsc.cumsum(lax.rev(v, (0,))), (0,))
        hit = (cum + sfx) >= k_splat
        pc = _popc(hit)
        jstar = jnp.maximum(pc - 1, 0)
        anyv = jnp.logical_and(pc > 0, found == 0)
        vj = _gat(v, jstar)
        sj = _gat(sfx, jstar)
        b_new = jnp.where(anyv, i * LANES + jstar, b)
        cg_new = jnp.where(anyv, cum + sj - vj, cg)
        cb_new = jnp.where(anyv, vj, cb)
        fnd = jnp.where(anyv, 1, found)
        cum_new = jnp.where(found == 0, cum + _gat(sfx, zerov), cum)
        return (cum_new, b_new, cg_new, cb_new, fnd)

    st = lax.fori_loop(
        0, niter, body, (zerov, zerov, zerov, zerov, zerov))
    return st[1], st[2], st[3]


def _clear_hist(hist_v, nvregs):
    zeros16 = jnp.zeros((LANES,), jnp.int32)

    @plsc.parallel_loop(0, nvregs, unroll=4)
    def _(i):
        hist_v[pl.ds(i * LANES, LANES)] = zeros16


def _radix_full(row_v, k_splat, hist_v, iota, ones16):
    _clear_hist(hist_v, 2048 // LANES)

    @plsc.parallel_loop(0, NV, unroll=8)
    def _(i):
        uv = _sortable(row_v[pl.ds(i * LANES, LANES)])
        plsc.addupdate_scatter(hist_v, [(uv >> 21) + 1024], ones16)

    nv1 = 2048 // LANES
    b1, cgt1, c1 = _scan_fixed(hist_v, nv1 - 1, k_splat, iota, nv1)
    hh1 = b1 - 1024

    _clear_hist(hist_v, 2048 // LANES)

    @plsc.parallel_loop(0, NV, unroll=8)
    def _(i):
        uv = _sortable(row_v[pl.ds(i * LANES, LANES)])
        act = (uv >> 21) == hh1
        plsc.addupdate_scatter(
            hist_v, [(uv >> 10) & 0x7FF], ones16, mask=act)

    k2 = k_splat - cgt1
    b2, cgt2, c2 = _scan_fixed(hist_v, nv1 - 1, k2, iota, nv1)
    pre22 = (hh1 << 11) | b2

    _clear_hist(hist_v, 1024 // LANES)

    @plsc.parallel_loop(0, NV, unroll=8)
    def _(i):
        uv = _sortable(row_v[pl.ds(i * LANES, LANES)])
        act = (uv >> 10) == pre22
        plsc.addupdate_scatter(hist_v, [uv & 0x3FF], ones16, mask=act)

    nv3 = 1024 // LANES
    k3 = k2 - cgt2
    b3, cgt3, ceq = _scan_fixed(hist_v, nv3 - 1, k3, iota, nv3)
    t = (pre22 << 10) | b3
    return t, k3 - cgt3, ceq


def _write_output(row_v, t, m, ceq):
    zf = jnp.float32(0.0)

    def out_simple(_):
        @plsc.parallel_loop(0, NV, unroll=8)
        def _(i):
            xv = row_v[pl.ds(i * LANES, LANES)]
            keep = _sortable(xv) >= t
            row_v[pl.ds(i * LANES, LANES)] = jnp.where(keep, xv, zf)

        return 0

    def out_ties(_):
        def ob(i, c):
            xv = row_v[pl.ds(i * LANES, LANES)]
            uv = _sortable(xv)
            eq = uv == t
            eqi = eq.astype(jnp.int32)
            pref = plsc.cumsum(eqi)
            keep = jnp.logical_or(
                uv > t, jnp.logical_and(eq, (pref + c) <= m))
            row_v[pl.ds(i * LANES, LANES)] = jnp.where(keep, xv, zf)
            return c + jnp.sum(eqi)

        lax.fori_loop(0, NV, ob, jnp.int32(0))
        return 0

    lax.cond(jnp.max(m) == jnp.max(ceq), out_simple, out_ties, 0)


def _select_and_mask(row_v, cand_v, hist_v, iota, ones16, hg):
    kk = jnp.int32(KTOP)
    kkv = jnp.full((LANES,), KTOP, jnp.int32)

    zero16 = jnp.zeros((LANES,), jnp.int32)
    base_idx = iota * CSTRIDE

    @plsc.parallel_loop(0, NV, unroll=8, carry=(zero16, zero16))
    def gt_c(i, st):
        gtv, cvec = st
        uv = _sortable(row_v[pl.ds(i * LANES, LANES)])
        hb = uv >> 21
        act = hb == hg
        plsc.store_scatter(cand_v, [base_idx + cvec], uv, mask=act)
        return (gtv + jnp.where(hb > hg, 1, 0),
                cvec + jnp.where(act, 1, 0))

    gtv, cvec = gt_c
    cnt_gt = jnp.sum(gtv)
    c1 = jnp.sum(cvec)
    jmax = jnp.max(cvec)
    valid = jnp.logical_and(
        jnp.logical_and(cnt_gt < kk, (cnt_gt + c1) >= kk), jmax <= JCAP)

    def fast(_):
        k1 = kkv - cnt_gt
        nh = 128 // LANES

        def stage(act_fn, bucket_fn, k_stage):
            _clear_hist(hist_v, nh)

            @plsc.parallel_loop(0, JCAP, unroll=4)
            def _(j):
                uv = plsc.load_gather(cand_v, [base_idx + j])
                ok = jnp.logical_and(j < cvec, act_fn(uv))
                plsc.addupdate_scatter(
                    hist_v, [bucket_fn(uv)], ones16, mask=ok)

            return _scan_fixed(hist_v, nh - 1, k_stage, iota, nh)

        p1, cg1, _ = stage(
            lambda uv: jnp.full((LANES,), True),
            lambda uv: (uv >> 14) & 0x7F, k1)
        k2 = k1 - cg1
        p2, cg2, _ = stage(
            lambda uv: ((uv >> 14) & 0x7F) == p1,
            lambda uv: (uv >> 7) & 0x7F, k2)
        p12 = (p1 << 7) | p2
        k3 = k2 - cg2
        p3, cg3, ceq = stage(
            lambda uv: ((uv >> 7) & 0x3FFF) == p12,
            lambda uv: uv & 0x7F, k3)
        t = (hg << 21) | (p12 << 7) | p3
        return t, k3 - cg3, ceq

    def classic(_):
        return _radix_full(row_v, kkv, hist_v, iota, ones16)

    t, m, ceq = lax.cond(valid, fast, classic, 0)
    _write_output(row_v, t, m, ceq)
    return t >> 21


def _topk_body(x_hbm, out_hbm, row_a, row_b, cand_v, hist_v,
               sem_ia, sem_ib, sem_oa, sem_ob):
    wid = lax.axis_index("s") * NC + lax.axis_index("c")
    base = wid * ROWS_PER_W
    iota = lax.iota(jnp.int32, LANES)
    ones16 = jnp.ones((LANES,), jnp.int32)
    npair = ROWS_PER_W // 2

    pltpu.async_copy(x_hbm.at[base], row_a, sem_ia)

    def per_pair(rr, carry):
        r0 = base + 2 * rr
        r1 = r0 + 1

        @pl.when(rr > 0)
        def _():
            pltpu.make_async_copy(row_b, out_hbm.at[r1 - 2], sem_ob).wait()

        pltpu.async_copy(x_hbm.at[r1], row_b, sem_ib)

        pltpu.make_async_copy(x_hbm.at[r0], row_a, sem_ia).wait()
        hg1 = _select_and_mask(row_a, cand_v, hist_v, iota, ones16, carry)
        pltpu.async_copy(row_a, out_hbm.at[r0], sem_oa)

        pltpu.make_async_copy(x_hbm.at[r1], row_b, sem_ib).wait()
        hg2 = _select_and_mask(row_b, cand_v, hist_v, iota, ones16, hg1)
        pltpu.async_copy(row_b, out_hbm.at[r1], sem_ob)

        @pl.when(rr < npair - 1)
        def _():
            pltpu.make_async_copy(row_a, out_hbm.at[r0], sem_oa).wait()
            pltpu.async_copy(x_hbm.at[r0 + 2], row_a, sem_ia)

        return hg2

    lax.fori_loop(0, npair, per_pair,
                  jnp.full((LANES,), -2048, jnp.int32))

    last = base + ROWS_PER_W - 1
    pltpu.make_async_copy(row_a, out_hbm.at[last - 1], sem_oa).wait()
    pltpu.make_async_copy(row_b, out_hbm.at[last], sem_ob).wait()


def kernel(x):
    mesh = plsc.VectorSubcoreMesh(
        core_axis_name="c", subcore_axis_name="s",
        num_cores=NC, num_subcores=NS)
    fn = functools.partial(
        pl.kernel,
        mesh=mesh,
        compiler_params=pltpu.CompilerParams(needs_layout_passes=False),
        out_type=jax.ShapeDtypeStruct((ROWS, COLS), jnp.float32),
        scratch_types=[
            pltpu.VMEM((COLS,), jnp.float32),
            pltpu.VMEM((COLS,), jnp.float32),
            pltpu.VMEM((LANES * CSTRIDE,), jnp.int32),
            pltpu.VMEM((2048,), jnp.int32),
            pltpu.SemaphoreType.DMA,
            pltpu.SemaphoreType.DMA,
            pltpu.SemaphoreType.DMA,
            pltpu.SemaphoreType.DMA,
        ],
    )(_topk_body)
    return fn(x)


# speedup vs baseline: 1.0227x; 1.0227x over previous
import functools

import jax
import jax.numpy as jnp
from jax import lax
from jax.experimental import pallas as pl
from jax.experimental.pallas import tpu as pltpu
from jax.experimental.pallas import tpu_sc as plsc

KTOP = 100
ROWS = 4096
COLS = 16384
LANES = 16
NV = COLS // LANES
NC = 2
NS = 16
NW = NC * NS
ROWS_PER_W = ROWS // NW
SAMPLE_STRIDE = 8
QLO = 13
CSTRIDE = 1025
JCAP = 32


def _sortable(xv):
    iv = lax.bitcast_convert_type(xv, jnp.int32)
    return jnp.where(iv < 0, iv ^ 0x7FFFFFFF, iv)


def _gat(vec, idx_splat):
    return vec.at[idx_splat].get(mode="promise_in_bounds")


def _popc(mask):
    return plsc.all_reduce_population_count(mask)


def _scan_fixed(hist_v, start_vreg, k_splat, iota, niter):
    zerov = jnp.zeros((LANES,), jnp.int32)

    def body(j, st):
        cum, b, cg, cb, found = st
        i = jnp.maximum(start_vreg - j, 0)
        v = hist_v[pl.ds(i * LANES, LANES)]
        sfx = lax.rev(plsc.cumsum(lax.rev(v, (0,))), (0,))
        hit = (cum + sfx) >= k_splat
        pc = _popc(hit)
        jstar = jnp.maximum(pc - 1, 0)
        anyv = jnp.logical_and(pc > 0, found == 0)
        vj = _gat(v, jstar)
        sj = _gat(sfx, jstar)
        b_new = jnp.where(anyv, i * LANES + jstar, b)
        cg_new = jnp.where(anyv, cum + sj - vj, cg)
        cb_new = jnp.where(anyv, vj, cb)
        fnd = jnp.where(anyv, 1, found)
        cum_new = jnp.where(found == 0, cum + _gat(sfx, zerov), cum)
        return (cum_new, b_new, cg_new, cb_new, fnd)

    st = lax.fori_loop(
        0, niter, body, (zerov, zerov, zerov, zerov, zerov))
    return st[1], st[2], st[3]


def _clear_hist(hist_v, nvregs):
    zeros16 = jnp.zeros((LANES,), jnp.int32)

    @plsc.parallel_loop(0, nvregs, unroll=4)
    def _(i):
        hist_v[pl.ds(i * LANES, LANES)] = zeros16


def _radix_full(row_v, k_splat, hist_v, iota, ones16):
    _clear_hist(hist_v, 2048 // LANES)

    @plsc.parallel_loop(0, NV, unroll=8)
    def _(i):
        uv = _sortable(row_v[pl.ds(i * LANES, LANES)])
        plsc.addupdate_scatter(hist_v, [(uv >> 21) + 1024], ones16)

    nv1 = 2048 // LANES
    b1, cgt1, c1 = _scan_fixed(hist_v, nv1 - 1, k_splat, iota, nv1)
    hh1 = b1 - 1024

    _clear_hist(hist_v, 2048 // LANES)

    @plsc.parallel_loop(0, NV, unroll=8)
    def _(i):
        uv = _sortable(row_v[pl.ds(i * LANES, LANES)])
        act = (uv >> 21) == hh1
        plsc.addupdate_scatter(
            hist_v, [(uv >> 10) & 0x7FF], ones16, mask=act)

    k2 = k_splat - cgt1
    b2, cgt2, c2 = _scan_fixed(hist_v, nv1 - 1, k2, iota, nv1)
    pre22 = (hh1 << 11) | b2

    _clear_hist(hist_v, 1024 // LANES)

    @plsc.parallel_loop(0, NV, unroll=8)
    def _(i):
        uv = _sortable(row_v[pl.ds(i * LANES, LANES)])
        act = (uv >> 10) == pre22
        plsc.addupdate_scatter(hist_v, [uv & 0x3FF], ones16, mask=act)

    nv3 = 1024 // LANES
    k3 = k2 - cgt2
    b3, cgt3, ceq = _scan_fixed(hist_v, nv3 - 1, k3, iota, nv3)
    t = (pre22 << 10) | b3
    return t, k3 - cgt3, ceq


def _write_output(row_v, t, m, ceq):
    zf = jnp.float32(0.0)

    def out_simple(_):
        @plsc.parallel_loop(0, NV, unroll=8)
        def _(i):
            xv = row_v[pl.ds(i * LANES, LANES)]
            keep = _sortable(xv) >= t
            row_v[pl.ds(i * LANES, LANES)] = jnp.where(keep, xv, zf)

        return 0

    def out_ties(_):
        def ob(i, c):
            xv = row_v[pl.ds(i * LANES, LANES)]
            uv = _sortable(xv)
            eq = uv == t
            eqi = eq.astype(jnp.int32)
            pref = plsc.cumsum(eqi)
            keep = jnp.logical_or(
                uv > t, jnp.logical_and(eq, (pref + c) <= m))
            row_v[pl.ds(i * LANES, LANES)] = jnp.where(keep, xv, zf)
            return c + jnp.sum(eqi)

        lax.fori_loop(0, NV, ob, jnp.int32(0))
        return 0

    lax.cond(jnp.max(m) == jnp.max(ceq), out_simple, out_ties, 0)


def _select_and_mask(row_v, cand_v, hist_v, iota, ones16):
    kk = jnp.int32(KTOP)
    kkv = jnp.full((LANES,), KTOP, jnp.int32)

    _clear_hist(hist_v, 2048 // LANES)

    @plsc.parallel_loop(
        0, NV // SAMPLE_STRIDE, unroll=8,
        carry=jnp.full((LANES,), -1, jnp.int32))
    def bmaxsv(i, vmax):
        uv = _sortable(row_v[pl.ds(i * SAMPLE_STRIDE * LANES, LANES)])
        b = (uv >> 21) + 1024
        plsc.addupdate_scatter(hist_v, [b], ones16)
        return jnp.maximum(vmax, b)

    bhi_s = jnp.max(bmaxsv)
    qlo = jnp.full((LANES,), QLO, jnp.int32)
    bg, _, _ = _scan_fixed(hist_v, bhi_s // LANES, qlo, iota, 4)
    hg = bg - 1024

    zero16 = jnp.zeros((LANES,), jnp.int32)
    base_idx = iota * CSTRIDE

    @plsc.parallel_loop(0, NV, unroll=8, carry=(zero16, zero16))
    def gt_c(i, st):
        gtv, cvec = st
        uv = _sortable(row_v[pl.ds(i * LANES, LANES)])
        hb = uv >> 21
        act = hb == hg
        plsc.store_scatter(cand_v, [base_idx + cvec], uv, mask=act)
        return (gtv + jnp.where(hb > hg, 1, 0),
                cvec + jnp.where(act, 1, 0))

    gtv, cvec = gt_c
    cnt_gt = jnp.sum(gtv)
    c1 = jnp.sum(cvec)
    jmax = jnp.max(cvec)
    valid = jnp.logical_and(
        jnp.logical_and(cnt_gt < kk, (cnt_gt + c1) >= kk), jmax <= JCAP)

    def fast(_):
        k1 = kkv - cnt_gt
        nh = 128 // LANES

        def stage(act_fn, bucket_fn, k_stage):
            _clear_hist(hist_v, nh)

            @plsc.parallel_loop(0, JCAP, unroll=4)
            def _(j):
                uv = plsc.load_gather(cand_v, [base_idx + j])
                ok = jnp.logical_and(j < cvec, act_fn(uv))
                plsc.addupdate_scatter(
                    hist_v, [bucket_fn(uv)], ones16, mask=ok)

            return _scan_fixed(hist_v, nh - 1, k_stage, iota, nh)

        p1, cg1, _ = stage(
            lambda uv: jnp.full((LANES,), True),
            lambda uv: (uv >> 14) & 0x7F, k1)
        k2 = k1 - cg1
        p2, cg2, _ = stage(
            lambda uv: ((uv >> 14) & 0x7F) == p1,
            lambda uv: (uv >> 7) & 0x7F, k2)
        p12 = (p1 << 7) | p2
        k3 = k2 - cg2
        p3, cg3, ceq = stage(
            lambda uv: ((uv >> 7) & 0x3FFF) == p12,
            lambda uv: uv & 0x7F, k3)
        t = (hg << 21) | (p12 << 7) | p3
        return t, k3 - cg3, ceq

    def classic(_):
        return _radix_full(row_v, kkv, hist_v, iota, ones16)

    t, m, ceq = lax.cond(valid, fast, classic, 0)
    _write_output(row_v, t, m, ceq)


def _topk_body(x_hbm, out_hbm, row_a, row_b, cand_v, hist_v,
               sem_ia, sem_ib, sem_oa, sem_ob):
    wid = lax.axis_index("s") * NC + lax.axis_index("c")
    base = wid * ROWS_PER_W
    iota = lax.iota(jnp.int32, LANES)
    ones16 = jnp.ones((LANES,), jnp.int32)
    npair = ROWS_PER_W // 2

    pltpu.async_copy(x_hbm.at[base], row_a, sem_ia)

    def per_pair(rr, carry):
        r0 = base + 2 * rr
        r1 = r0 + 1

        @pl.when(rr > 0)
        def _():
            pltpu.make_async_copy(row_b, out_hbm.at[r1 - 2], sem_ob).wait()

        pltpu.async_copy(x_hbm.at[r1], row_b, sem_ib)

        pltpu.make_async_copy(x_hbm.at[r0], row_a, sem_ia).wait()
        _select_and_mask(row_a, cand_v, hist_v, iota, ones16)
        pltpu.async_copy(row_a, out_hbm.at[r0], sem_oa)

        pltpu.make_async_copy(x_hbm.at[r1], row_b, sem_ib).wait()
        _select_and_mask(row_b, cand_v, hist_v, iota, ones16)
        pltpu.async_copy(row_b, out_hbm.at[r1], sem_ob)

        @pl.when(rr < npair - 1)
        def _():
            pltpu.make_async_copy(row_a, out_hbm.at[r0], sem_oa).wait()
            pltpu.async_copy(x_hbm.at[r0 + 2], row_a, sem_ia)

        return carry

    lax.fori_loop(0, npair, per_pair, 0)

    last = base + ROWS_PER_W - 1
    pltpu.make_async_copy(row_a, out_hbm.at[last - 1], sem_oa).wait()
    pltpu.make_async_copy(row_b, out_hbm.at[last], sem_ob).wait()


def kernel(x):
    mesh = plsc.VectorSubcoreMesh(
        core_axis_name="c", subcore_axis_name="s",
        num_cores=NC, num_subcores=NS)
    fn = functools.partial(
        pl.kernel,
        mesh=mesh,
        compiler_params=pltpu.CompilerParams(needs_layout_passes=False),
        out_type=jax.ShapeDtypeStruct((ROWS, COLS), jnp.float32),
        scratch_types=[
            pltpu.VMEM((COLS,), jnp.float32),
            pltpu.VMEM((COLS,), jnp.float32),
            pltpu.VMEM((LANES * CSTRIDE,), jnp.int32),
            pltpu.VMEM((2048,), jnp.int32),
            pltpu.SemaphoreType.DMA,
            pltpu.SemaphoreType.DMA,
            pltpu.SemaphoreType.DMA,
            pltpu.SemaphoreType.DMA,
        ],
    )(_topk_body)
    return fn(x)
